# baseline (speedup 1.0000x reference)
import numpy as np

import jax
import jax.numpy as jnp
from jax import lax
from jax.experimental import pallas as pl
from jax.experimental.pallas import tpu as pltpu
from jax.experimental.pallas import tpu_sc as plsc

B = 4096
N_DENSE = 13
F = 26
V = 100000
E = 16
EPS = 1e-3

LANES = 16
NC, NS = 2, 16
NW = NC * NS
BW = B // NW
ROWS = BW * F
CHUNK = 128
NCHUNK = ROWS // CHUNK
KPC = CHUNK // LANES

_FIELD_OFFS = ((np.arange(ROWS, dtype=np.int64) % F) * V).astype(np.int32).reshape(NCHUNK, CHUNK)


def _fm_pool_body(sparse_hbm, offs_hbm, table_hbm, out_hbm, idx_v, offs_v, rows_v, out_v, sem):
    wid = lax.axis_index("s") * NC + lax.axis_index("c")
    pltpu.sync_copy(sparse_hbm.at[wid], idx_v)
    pltpu.sync_copy(offs_hbm, offs_v)

    def adj(i, c):
        j = i // KPC
        k = i - j * KPC
        sl = pl.ds(k * LANES, LANES)
        idx_v[j, sl] = idx_v[j, sl] + offs_v[j, sl]
        return c

    lax.fori_loop(0, NCHUNK * KPC, adj, 0)

    copies = [
        pltpu.async_copy(table_hbm.at[idx_v.at[j]], rows_v.at[pl.ds(j * CHUNK, CHUNK)], sem)
        for j in range(NCHUNK)
    ]
    for c in copies:
        c.wait()

    def body(b, c):
        base = b * F
        s = jnp.zeros((LANES,), jnp.float32)
        ss = jnp.zeros((LANES,), jnp.float32)
        for f in range(F):
            r = rows_v[base + f, :]
            s = s + r
            ss = ss + r * r
        out_v[b, :] = 0.5 * (s * s - ss)
        return c

    lax.fori_loop(0, BW, body, 0)
    pltpu.sync_copy(out_v, out_hbm.at[pl.ds(wid * BW, BW)])


def _fm_pool(sparse2d, offs, table2d):
    mesh = plsc.VectorSubcoreMesh(core_axis_name="c", subcore_axis_name="s")
    return pl.kernel(
        _fm_pool_body,
        out_type=jax.ShapeDtypeStruct((B, E), jnp.float32),
        mesh=mesh,
        compiler_params=pltpu.CompilerParams(use_tc_tiling_on_sc=False),
        scratch_types=[
            pltpu.VMEM((NCHUNK, CHUNK), jnp.int32),
            pltpu.VMEM((NCHUNK, CHUNK), jnp.int32),
            pltpu.VMEM((ROWS, E), jnp.float32),
            pltpu.VMEM((BW, E), jnp.float32),
            pltpu.SemaphoreType.DMA,
        ],
    )(sparse2d, offs, table2d)


def _mlp_body(d_ref, e_ref, gd_ref, ge_ref, bd_ref, be_ref,
              w1a_ref, w1b_ref, b1_ref, w2_ref, b2_ref, w3_ref, b3_ref,
              wo_ref, bo_ref, o_ref):
    d = d_ref[:]
    e = e_ref[:]
    md = jnp.mean(d, axis=0, keepdims=True)
    dc = d - md
    vd = jnp.mean(dc * dc, axis=0, keepdims=True)
    dn = gd_ref[:] * dc * lax.rsqrt(vd + EPS) + bd_ref[:]
    me = jnp.mean(e, axis=0, keepdims=True)
    ec = e - me
    ve = jnp.mean(ec * ec, axis=0, keepdims=True)
    en = ge_ref[:] * ec * lax.rsqrt(ve + EPS) + be_ref[:]
    h = (jnp.dot(dn, w1a_ref[:], preferred_element_type=jnp.float32)
         + jnp.dot(en, w1b_ref[:], preferred_element_type=jnp.float32)
         + b1_ref[:])
    h = jnp.maximum(h, 0.0)
    h = jnp.maximum(jnp.dot(h, w2_ref[:], preferred_element_type=jnp.float32) + b2_ref[:], 0.0)
    h = jnp.maximum(jnp.dot(h, w3_ref[:], preferred_element_type=jnp.float32) + b3_ref[:], 0.0)
    o_ref[:] = jax.nn.sigmoid(
        jnp.dot(h, wo_ref[:], preferred_element_type=jnp.float32) + bo_ref[:])


def kernel(dense_inputs, sparse_inputs, tables, gamma, beta, W1, b1, W2, b2, W3, b3, Wout, bout):
    table2d = tables.reshape(F * V, E)
    sparse2d = sparse_inputs.astype(jnp.int32).reshape(NW, NCHUNK, CHUNK)
    offs = jnp.asarray(_FIELD_OFFS)

    bi = _fm_pool(sparse2d, offs, table2d)

    gd = gamma[:N_DENSE].reshape(1, N_DENSE)
    ge = gamma[N_DENSE:].reshape(1, E)
    bd = beta[:N_DENSE].reshape(1, N_DENSE)
    be = beta[N_DENSE:].reshape(1, E)
    w1a = W1[:N_DENSE]
    w1b = W1[N_DENSE:]

    out = pl.pallas_call(
        _mlp_body,
        out_shape=jax.ShapeDtypeStruct((B, 1), jnp.float32),
    )(dense_inputs, bi, gd, ge, bd, be,
      w1a, w1b, b1.reshape(1, -1), W2, b2.reshape(1, -1), W3, b3.reshape(1, -1),
      Wout, bout.reshape(1, -1))
    return out
